# Initial kernel scaffold; baseline (speedup 1.0000x reference)
import jax
import jax.numpy as jnp
from jax.experimental import pallas as pl


def kernel(inputs, embedding):
    raise NotImplementedError("write your pallas kernel here")


# speedup vs baseline: 1.1731x; 1.1731x over previous
import functools

import jax
import jax.numpy as jnp
from jax.experimental import pallas as pl
from jax.experimental.pallas import tpu as pltpu

BETA = 0.25


def _vq_body(x_ref, e_ref, out_ref, loss_ref, esq_ref, *, RB, KB, NK, D, K, scale):
    i = pl.program_id(0)

    @pl.when(i == 0)
    def _():
        e = e_ref[...]
        esq_ref[...] = jnp.sum(e * e, axis=0, keepdims=True)
        loss_ref[...] = jnp.zeros_like(loss_ref)

    x = x_ref[...]
    xb = x.astype(jnp.bfloat16)
    xsq = jnp.sum(x * x, axis=1, keepdims=True)

    def dist_step(kb, carry):
        rmin, ridx = carry
        off = kb * KB
        e_blk = e_ref[:, pl.ds(off, KB)].astype(jnp.bfloat16)
        sim = jax.lax.dot_general(
            xb, e_blk, (((1,), (0,)), ((), ())),
            preferred_element_type=jnp.float32)
        d = (xsq + esq_ref[:, pl.ds(off, KB)]) - 2.0 * sim
        m = jnp.min(d, axis=1, keepdims=True)
        iota = jax.lax.broadcasted_iota(jnp.int32, (RB, KB), 1) + off
        bidx = jnp.min(jnp.where(d == m, iota, K), axis=1, keepdims=True)
        better = m < rmin
        return jnp.where(better, m, rmin), jnp.where(better, bidx, ridx)

    rmin0 = jnp.full((RB, 1), jnp.inf, jnp.float32)
    ridx0 = jnp.zeros((RB, 1), jnp.int32)
    _, ridx = jax.lax.fori_loop(0, NK, dist_step, (rmin0, ridx0))

    def onehot_step(kb, qacc):
        off = kb * KB
        e_blk = e_ref[:, pl.ds(off, KB)].astype(jnp.bfloat16)
        iota = jax.lax.broadcasted_iota(jnp.int32, (RB, KB), 1) + off
        oh = (iota == ridx).astype(jnp.bfloat16)
        return qacc + jax.lax.dot_general(
            oh, e_blk, (((1,), (1,)), ((), ())),
            preferred_element_type=jnp.float32)

    q = jax.lax.fori_loop(0, NK, onehot_step, jnp.zeros((RB, D), jnp.float32))
    diff = q - x
    out_ref[...] = x + diff
    loss_ref[...] += jnp.sum(diff * diff, axis=(0, 1), keepdims=True) * scale


def kernel(inputs, embedding):
    orig_shape = inputs.shape
    x = inputs.reshape(-1, orig_shape[-1])
    N, D = x.shape
    K = embedding.shape[1]
    RB = 512 if N % 512 == 0 else N
    KB = 2048 if K % 2048 == 0 else K
    NR, NK = N // RB, K // KB
    scale = (1.0 + BETA) / float(inputs.size)

    body = functools.partial(_vq_body, RB=RB, KB=KB, NK=NK, D=D, K=K, scale=scale)
    out, loss = pl.pallas_call(
        body,
        grid=(NR,),
        in_specs=[
            pl.BlockSpec((RB, D), lambda i: (i, 0)),
            pl.BlockSpec((D, K), lambda i: (0, 0)),
        ],
        out_specs=[
            pl.BlockSpec((RB, D), lambda i: (i, 0)),
            pl.BlockSpec((1, 1), lambda i: (0, 0)),
        ],
        out_shape=[
            jax.ShapeDtypeStruct((N, D), jnp.float32),
            jax.ShapeDtypeStruct((1, 1), jnp.float32),
        ],
        scratch_shapes=[pltpu.VMEM((1, K), jnp.float32)],
        compiler_params=pltpu.CompilerParams(
            dimension_semantics=("arbitrary",)),
    )(x, embedding)
    return out.reshape(orig_shape), loss.reshape(())
